# baseline (speedup 1.0000x reference)
import jax
import jax.numpy as jnp
from jax.experimental import pallas as pl
from jax.experimental.pallas import tpu as pltpu

SIZE = 128
ROW_BLOCK = 128
PIPE = 8


def _snn_body(xt_ref, w_ref, out_ref, a_scr, sx_scr):
    xt = xt_ref[...]
    w = w_ref[...]
    jiota = jax.lax.broadcasted_iota(jnp.int32, (SIZE, 1), 0)

    def rank_step(k, rank_t):
        xk = xt_ref[pl.ds(k, 1), :]
        lt = xk < xt
        tie = jnp.logical_and(xk == xt, k < jiota)
        return rank_t + jnp.logical_or(lt, tie).astype(jnp.float32)

    rank_t = jax.lax.fori_loop(
        0, SIZE, rank_step, jnp.zeros((SIZE, ROW_BLOCK), jnp.float32))

    rank_i = rank_t.astype(jnp.int32)

    iota3 = jax.lax.broadcasted_iota(jnp.int32, (SIZE, SIZE, ROW_BLOCK), 0)
    eq3 = iota3 == rank_i[None, :, :]
    sx_scr[...] = jnp.sum(jnp.where(eq3, xt[None, :, :], 0.0),
                          axis=1, keepdims=True)

    def gather_dot(i):
        p = (rank_i == i).astype(jnp.float32)
        a_scr[i] = jax.lax.dot_general(
            w, p, (((1,), (0,)), ((), ())),
            precision=jax.lax.Precision.HIGHEST,
            preferred_element_type=jnp.float32)

    def prologue(i, c):
        gather_dot(i)
        return c

    jax.lax.fori_loop(0, PIPE, prologue, 0)

    def scan_math(m, state):
        c_w, c_wx, num, den = state
        a = a_scr[m]
        sx = sx_scr[m]
        c_w = c_w + a
        c_wx = c_wx + a * sx
        denom = c_w - 1.0
        cond = jnp.logical_and(c_wx > sx * denom, c_w > 1.0)
        newly = jnp.logical_and(cond, den == 0.0)
        num = jnp.where(newly, c_wx, num)
        den = jnp.where(newly, denom, den)
        return c_w, c_wx, num, den

    def main_body(m, state):
        gather_dot(m + PIPE)
        return scan_math(m, state)

    zeros = jnp.zeros((SIZE, ROW_BLOCK), jnp.float32)
    state = (zeros, zeros, zeros, zeros)
    state = jax.lax.fori_loop(0, SIZE - PIPE, main_body, state)
    state = jax.lax.fori_loop(SIZE - PIPE, SIZE, scan_math, state)
    _, _, num, den = state
    out_ref[...] = jnp.where(den == 0.0, 1e10, num / den)


@jax.jit
def kernel(input, w):
    x = input
    batch = x.shape[0]
    out = pl.pallas_call(
        _snn_body,
        grid=(batch // ROW_BLOCK,),
        in_specs=[
            pl.BlockSpec((SIZE, ROW_BLOCK), lambda g: (0, g)),
            pl.BlockSpec((SIZE, SIZE), lambda g: (0, 0)),
        ],
        out_specs=pl.BlockSpec((SIZE, ROW_BLOCK), lambda g: (0, g)),
        out_shape=jax.ShapeDtypeStruct((SIZE, batch), jnp.float32),
        scratch_shapes=[
            pltpu.VMEM((SIZE, SIZE, ROW_BLOCK), jnp.float32),
            pltpu.VMEM((SIZE, 1, ROW_BLOCK), jnp.float32),
        ],
    )(x.T, w)
    return out.T[:, :, None]


# speedup vs baseline: 1.0139x; 1.0139x over previous
import jax
import jax.numpy as jnp
from jax.experimental import pallas as pl
from jax.experimental.pallas import tpu as pltpu

SIZE = 128
ROW_BLOCK = 128
PIPE = 1


def _snn_body(xt_ref, w_ref, out_ref, sx_scr):
    xt = xt_ref[...]
    w = w_ref[...]
    jiota = jax.lax.broadcasted_iota(jnp.int32, (SIZE, 1), 0)

    def rank_step(k, rank_t):
        xk = xt_ref[pl.ds(k, 1), :]
        lt = xk < xt
        tie = jnp.logical_and(xk == xt, k < jiota)
        return rank_t + jnp.logical_or(lt, tie).astype(jnp.float32)

    rank_t = jax.lax.fori_loop(
        0, SIZE, rank_step, jnp.zeros((SIZE, ROW_BLOCK), jnp.float32))

    rank_i = rank_t.astype(jnp.int32)

    iota3 = jax.lax.broadcasted_iota(jnp.int32, (SIZE, SIZE, ROW_BLOCK), 0)
    eq3 = iota3 == rank_i[None, :, :]
    sx_scr[...] = jnp.sum(jnp.where(eq3, xt[None, :, :], 0.0),
                          axis=1, keepdims=True)

    def gather_dot(i):
        p = (rank_i == i).astype(jnp.float32)
        return jax.lax.dot_general(
            w, p, (((1,), (0,)), ((), ())),
            precision=jax.lax.Precision.HIGHEST,
            preferred_element_type=jnp.float32)

    def main_body(m, state):
        c_w, c_wx, num, den, a = state
        a_next = gather_dot(m + 1)
        sx = sx_scr[m]
        c_w = c_w + a
        c_wx = c_wx + a * sx
        denom = c_w - 1.0
        cond = jnp.logical_and(c_wx > sx * denom, c_w > 1.0)
        newly = jnp.logical_and(cond, den == 0.0)
        num = jnp.where(newly, c_wx, num)
        den = jnp.where(newly, denom, den)
        return c_w, c_wx, num, den, a_next

    zeros = jnp.zeros((SIZE, ROW_BLOCK), jnp.float32)
    state = (zeros, zeros, zeros, zeros, gather_dot(jnp.int32(0)))
    state = jax.lax.fori_loop(0, SIZE, main_body, state)
    _, _, num, den, _ = state
    out_ref[...] = jnp.where(den == 0.0, 1e10, num / den)


@jax.jit
def kernel(input, w):
    x = input
    batch = x.shape[0]
    out = pl.pallas_call(
        _snn_body,
        grid=(batch // ROW_BLOCK,),
        in_specs=[
            pl.BlockSpec((SIZE, ROW_BLOCK), lambda g: (0, g)),
            pl.BlockSpec((SIZE, SIZE), lambda g: (0, 0)),
        ],
        out_specs=pl.BlockSpec((SIZE, ROW_BLOCK), lambda g: (0, g)),
        out_shape=jax.ShapeDtypeStruct((SIZE, batch), jnp.float32),
        scratch_shapes=[
            pltpu.VMEM((SIZE, 1, ROW_BLOCK), jnp.float32),
        ],
    )(x.T, w)
    return out.T[:, :, None]


# speedup vs baseline: 1.0545x; 1.0401x over previous
import jax
import jax.numpy as jnp
from jax.experimental import pallas as pl
from jax.experimental.pallas import tpu as pltpu

SIZE = 128
ROW_BLOCK = 128
PIPE = 1


def _snn_body(xt_ref, w_ref, out_ref, sx_scr, num_scr, den_scr):
    xt = xt_ref[...]
    w = w_ref[...]
    jiota = jax.lax.broadcasted_iota(jnp.int32, (SIZE, 1), 0)

    def rank_step(k, rank_t):
        xk = xt_ref[pl.ds(k, 1), :]
        lt = xk < xt
        tie = jnp.logical_and(xk == xt, k < jiota)
        return rank_t + jnp.logical_or(lt, tie).astype(jnp.float32)

    rank_t = jax.lax.fori_loop(
        0, SIZE, rank_step, jnp.zeros((SIZE, ROW_BLOCK), jnp.float32))

    rank_i = rank_t.astype(jnp.int32)

    iota3 = jax.lax.broadcasted_iota(jnp.int32, (SIZE, SIZE, ROW_BLOCK), 0)
    eq3 = iota3 == rank_i[None, :, :]
    sx_scr[...] = jnp.sum(jnp.where(eq3, xt[None, :, :], 0.0),
                          axis=1, keepdims=True)

    def gather_dot(i):
        p = (rank_i == i).astype(jnp.float32)
        return jax.lax.dot_general(
            w, p, (((1,), (0,)), ((), ())),
            precision=jax.lax.Precision.HIGHEST,
            preferred_element_type=jnp.float32)

    zeros = jnp.zeros((SIZE, ROW_BLOCK), jnp.float32)
    num_scr[...] = zeros
    den_scr[...] = zeros

    def main_body(m, state):
        c_w, c_wx, a = state
        a_next = gather_dot(m + 1)
        sx = sx_scr[m]
        c_w = c_w + a
        c_wx = c_wx + a * sx
        denom = c_w - 1.0
        cond = jnp.logical_and(c_wx > sx * denom, c_w > 1.0)
        num = num_scr[...]
        newly = jnp.logical_and(cond, num == 0.0)
        num_scr[...] = jnp.where(newly, c_wx, num)
        den_scr[...] = jnp.where(newly, denom, den_scr[...])
        return c_w, c_wx, a_next

    state = (zeros, zeros, gather_dot(jnp.int32(0)))
    jax.lax.fori_loop(0, SIZE, main_body, state)
    num = num_scr[...]
    out_ref[...] = jnp.where(num == 0.0, 1e10, num / den_scr[...])


@jax.jit
def kernel(input, w):
    x = input
    batch = x.shape[0]
    out = pl.pallas_call(
        _snn_body,
        grid=(batch // ROW_BLOCK,),
        in_specs=[
            pl.BlockSpec((SIZE, ROW_BLOCK), lambda g: (0, g)),
            pl.BlockSpec((SIZE, SIZE), lambda g: (0, 0)),
        ],
        out_specs=pl.BlockSpec((SIZE, ROW_BLOCK), lambda g: (0, g)),
        out_shape=jax.ShapeDtypeStruct((SIZE, batch), jnp.float32),
        scratch_shapes=[
            pltpu.VMEM((SIZE, 1, ROW_BLOCK), jnp.float32),
            pltpu.VMEM((SIZE, ROW_BLOCK), jnp.float32),
            pltpu.VMEM((SIZE, ROW_BLOCK), jnp.float32),
        ],
    )(x.T, w)
    return out.T[:, :, None]


# speedup vs baseline: 2.0684x; 1.9614x over previous
import jax
import jax.numpy as jnp
from jax.experimental import pallas as pl
from jax.experimental.pallas import tpu as pltpu

SIZE = 128
ROW_BLOCK = 128
PIPE = 1


def _snn_body(xt_ref, w_ref, out_ref, sx_scr, num_scr, den_scr):
    xt = xt_ref[...]
    w = w_ref[...]
    jiota = jax.lax.broadcasted_iota(jnp.int32, (SIZE, 1), 0)

    def rank_step(k, rank_t):
        xk = xt_ref[pl.ds(k, 1), :]
        lt = xk < xt
        tie = jnp.logical_and(xk == xt, k < jiota)
        return rank_t + jnp.logical_or(lt, tie).astype(jnp.float32)

    rank_t = jax.lax.fori_loop(
        0, SIZE, rank_step, jnp.zeros((SIZE, ROW_BLOCK), jnp.float32),
        unroll=4)

    rank_i = rank_t.astype(jnp.int32)

    iota3 = jax.lax.broadcasted_iota(jnp.int32, (SIZE, SIZE, ROW_BLOCK), 0)
    eq3 = iota3 == rank_i[None, :, :]
    sx_scr[...] = jnp.sum(jnp.where(eq3, xt[None, :, :], 0.0),
                          axis=1, keepdims=True)

    def gather_dot(i):
        p = (rank_i == i).astype(jnp.float32)
        return jax.lax.dot_general(
            w, p, (((1,), (0,)), ((), ())),
            precision=jax.lax.Precision.HIGHEST,
            preferred_element_type=jnp.float32)

    zeros = jnp.zeros((SIZE, ROW_BLOCK), jnp.float32)
    num_scr[...] = zeros
    den_scr[...] = zeros

    def main_body(m, state):
        c_w, c_wx, a = state
        a_next = gather_dot(m + 1)
        sx = sx_scr[m]
        c_w = c_w + a
        c_wx = c_wx + a * sx
        denom = c_w - 1.0
        cond = jnp.logical_and(c_wx > sx * denom, c_w > 1.0)
        num = num_scr[...]
        newly = jnp.logical_and(cond, num == 0.0)
        num_scr[...] = jnp.where(newly, c_wx, num)
        den_scr[...] = jnp.where(newly, denom, den_scr[...])
        return c_w, c_wx, a_next

    state = (zeros, zeros, gather_dot(jnp.int32(0)))
    jax.lax.fori_loop(0, SIZE, main_body, state, unroll=4)
    num = num_scr[...]
    out_ref[...] = jnp.where(num == 0.0, 1e10, num / den_scr[...])


@jax.jit
def kernel(input, w):
    x = input
    batch = x.shape[0]
    out = pl.pallas_call(
        _snn_body,
        grid=(batch // ROW_BLOCK,),
        in_specs=[
            pl.BlockSpec((SIZE, ROW_BLOCK), lambda g: (0, g)),
            pl.BlockSpec((SIZE, SIZE), lambda g: (0, 0)),
        ],
        out_specs=pl.BlockSpec((SIZE, ROW_BLOCK), lambda g: (0, g)),
        out_shape=jax.ShapeDtypeStruct((SIZE, batch), jnp.float32),
        scratch_shapes=[
            pltpu.VMEM((SIZE, 1, ROW_BLOCK), jnp.float32),
            pltpu.VMEM((SIZE, ROW_BLOCK), jnp.float32),
            pltpu.VMEM((SIZE, ROW_BLOCK), jnp.float32),
        ],
    )(x.T, w)
    return out.T[:, :, None]
